# Initial kernel scaffold; baseline (speedup 1.0000x reference)
import jax
import jax.numpy as jnp
from jax.experimental import pallas as pl


def kernel(x, beta, G, eps):
    raise NotImplementedError("write your pallas kernel here")


# speedup vs baseline: 1.8275x; 1.8275x over previous
import functools

import jax
import jax.numpy as jnp
import numpy as np
from jax import lax
from jax.experimental import pallas as pl
from jax.experimental.pallas import tpu as pltpu
from jax.experimental.pallas import tpu_sc as plsc

TINY = float(np.finfo(np.float32).eps)
Q = 4.0
N_ROWS = 256 * 576
NW = 32
ROWS_PER_W = N_ROWS // NW
GROUPS = ROWS_PER_W // 16


def _cround(v):
    s = jnp.sign(v)
    z = (v - s * TINY) + 0.5
    fi = z.astype(jnp.int32).astype(jnp.float32)
    return jnp.where(fi > z, fi - 1.0, fi)


def _gx(x, f):
    delta = [jnp.abs(x[j] - f[j]) for j in range(8)]
    m = delta[0]
    k = jnp.zeros_like(delta[0], dtype=jnp.int32)
    xk = x[0]
    fk = f[0]
    for j in range(1, 8):
        gt = delta[j] > m
        m = jnp.where(gt, delta[j], m)
        k = jnp.where(gt, j, k)
        xk = jnp.where(gt, x[j], xk)
        fk = jnp.where(gt, f[j], fk)
    pos = jnp.where(fk < xk, fk + 1.0, fk - 1.0)
    neg = jnp.where(fk <= xk, fk + 1.0, fk - 1.0)
    nk = jnp.where(xk >= 0, pos, neg)
    return [jnp.where(k == j, nk, f[j]) for j in range(8)]


def _parity_even(s):
    return (s.astype(jnp.int32) & 1) == 0


def _cp8(x):
    f = [_cround(xj) for xj in x]
    s0 = f[0]
    for j in range(1, 8):
        s0 = s0 + f[j]
    even0 = _parity_even(s0)
    g0 = _gx(x, f)
    y0 = [jnp.where(even0, f[j], g0[j]) for j in range(8)]
    xs = [xj - 0.5 for xj in x]
    fs = [_cround(xj) for xj in xs]
    s1 = fs[0]
    for j in range(1, 8):
        s1 = s1 + fs[j]
    even1 = _parity_even(s1)
    g1 = _gx(xs, fs)
    y1 = [jnp.where(even1, fs[j], g1[j]) + 0.5 for j in range(8)]
    r0 = x[0] - y0[0]
    r1 = x[0] - y1[0]
    d0 = r0 * r0
    d1 = r1 * r1
    for j in range(1, 8):
        r0 = x[j] - y0[j]
        r1 = x[j] - y1[j]
        d0 = d0 + r0 * r0
        d1 = d1 + r1 * r1
    c = d0 < d1
    return [jnp.where(c, y0[j], y1[j]) for j in range(8)]


def _quantize16(xs, e):
    xl = xs
    bs = []
    for _ in range(2):
        t = [xl[j] + e[j] for j in range(8)]
        cpv = _cp8(t)
        y = [None] * 8
        y[0] = cpv[0] * 0.5
        for k in range(1, 7):
            y[k] = cpv[k] + y[k - 1]
        ssum = y[0]
        for k in range(1, 7):
            ssum = ssum + y[k]
        y[7] = cpv[7] * 2.0 - ssum
        bs.append([_cround(jnp.fmod(y[j], Q)) for j in range(8)])
        xl = [cpv[j] * 0.25 for j in range(8)]
    xh = [jnp.zeros_like(xs[0]) for _ in range(8)]
    for i, b in enumerate(bs):
        pt = [None] * 8
        pt[0] = 2.0 * b[0]
        for k in range(1, 7):
            pt[k] = b[k] - b[k - 1]
        sb = b[0]
        for k in range(1, 8):
            sb = sb + b[k]
        pt[7] = 0.5 * sb
        u = [pt[j] * 0.25 for j in range(8)]
        c = _cp8(u)
        scale = Q ** i
        xh = [xh[j] + scale * (pt[j] - Q * c[j]) for j in range(8)]
    return xh


def _body(xf_hbm, beta_hbm, eps_hbm, out_hbm, in_v, out_v, beta_v, eps_v):
    wid = lax.axis_index("s") * 2 + lax.axis_index("c")
    base = wid * ROWS_PER_W * 8
    pltpu.sync_copy(xf_hbm.at[pl.ds(base, ROWS_PER_W * 8)], in_v)
    pltpu.sync_copy(beta_hbm, beta_v)
    pltpu.sync_copy(eps_hbm, eps_v)
    bv = beta_v[...]
    e = [eps_v[j] for j in range(8)]
    lanes = lax.iota(jnp.int32, 16)
    cols = [lanes * 8 + j for j in range(8)]

    def group(g, carry):
        goff = g * 128
        idx = [cols[j] + goff for j in range(8)]
        xs = [plsc.load_gather(in_v, [idx[j]]) / bv for j in range(8)]
        xh = _quantize16(xs, e)
        for j in range(8):
            plsc.store_scatter(out_v, [idx[j]], xh[j] * bv)
        return carry

    lax.fori_loop(0, GROUPS, group, 0)
    pltpu.sync_copy(out_v, out_hbm.at[pl.ds(base, ROWS_PER_W * 8)])


def kernel(x, beta, G, eps):
    orig = x.shape
    xf = x.reshape(-1)
    beta_b = jnp.broadcast_to(jnp.asarray(beta, jnp.float32).reshape(1), (16,))
    eps_b = jnp.broadcast_to(jnp.asarray(eps, jnp.float32).reshape(8, 1),
                             (8, 16))
    mesh = plsc.VectorSubcoreMesh(core_axis_name="c", subcore_axis_name="s")
    run = functools.partial(
        pl.kernel,
        out_type=jax.ShapeDtypeStruct((N_ROWS * 8,), jnp.float32),
        mesh=mesh,
        compiler_params=pltpu.CompilerParams(needs_layout_passes=False),
        scratch_types=[
            pltpu.VMEM((ROWS_PER_W * 8,), jnp.float32),
            pltpu.VMEM((ROWS_PER_W * 8,), jnp.float32),
            pltpu.VMEM((16,), jnp.float32),
            pltpu.VMEM((8, 16), jnp.float32),
        ],
    )(_body)
    out = run(xf, beta_b, eps_b)
    return out.reshape(orig)


# speedup vs baseline: 1.9420x; 1.0627x over previous
import functools

import jax
import jax.numpy as jnp
import numpy as np
from jax import lax
from jax.experimental import pallas as pl
from jax.experimental.pallas import tpu as pltpu
from jax.experimental.pallas import tpu_sc as plsc

TINY = float(np.finfo(np.float32).eps)
Q = 4.0
N_ROWS = 256 * 576
NW = 32
ROWS_PER_W = N_ROWS // NW
GROUPS = ROWS_PER_W // 16


_SIGN_BIT = np.int32(np.uint32(0x80000000))
_TINY_BITS = int(np.float32(TINY).view(np.int32))


def _cround(v):
    vb = lax.bitcast_convert_type(v, jnp.int32)
    t = lax.bitcast_convert_type((vb & _SIGN_BIT) | _TINY_BITS, jnp.float32)
    z = (v - t) + 0.5
    fi = z.astype(jnp.int32).astype(jnp.float32)
    return jnp.where(fi > z, fi - 1.0, fi)


def _gx(x, f):
    delta = [jnp.abs(x[j] - f[j]) for j in range(8)]
    m = delta[0]
    k = jnp.zeros_like(delta[0], dtype=jnp.int32)
    xk = x[0]
    fk = f[0]
    for j in range(1, 8):
        gt = delta[j] > m
        m = jnp.where(gt, delta[j], m)
        k = jnp.where(gt, j, k)
        xk = jnp.where(gt, x[j], xk)
        fk = jnp.where(gt, f[j], fk)
    pos = jnp.where(fk < xk, fk + 1.0, fk - 1.0)
    neg = jnp.where(fk <= xk, fk + 1.0, fk - 1.0)
    nk = jnp.where(xk >= 0, pos, neg)
    return [jnp.where(k == j, nk, f[j]) for j in range(8)]


def _parity_even(s):
    return (s.astype(jnp.int32) & 1) == 0


def _cp8(x):
    f = [_cround(xj) for xj in x]
    s0 = f[0]
    for j in range(1, 8):
        s0 = s0 + f[j]
    even0 = _parity_even(s0)
    g0 = _gx(x, f)
    y0 = [jnp.where(even0, f[j], g0[j]) for j in range(8)]
    xs = [xj - 0.5 for xj in x]
    fs = [_cround(xj) for xj in xs]
    s1 = fs[0]
    for j in range(1, 8):
        s1 = s1 + fs[j]
    even1 = _parity_even(s1)
    g1 = _gx(xs, fs)
    y1 = [jnp.where(even1, fs[j], g1[j]) + 0.5 for j in range(8)]
    r0 = x[0] - y0[0]
    r1 = x[0] - y1[0]
    d0 = r0 * r0
    d1 = r1 * r1
    for j in range(1, 8):
        r0 = x[j] - y0[j]
        r1 = x[j] - y1[j]
        d0 = d0 + r0 * r0
        d1 = d1 + r1 * r1
    c = d0 < d1
    return [jnp.where(c, y0[j], y1[j]) for j in range(8)]


def _quantize16(xs, e):
    xl = xs
    bs = []
    for _ in range(2):
        t = [xl[j] + e[j] for j in range(8)]
        cpv = _cp8(t)
        y = [None] * 8
        y[0] = cpv[0] * 0.5
        for k in range(1, 7):
            y[k] = cpv[k] + y[k - 1]
        ssum = y[0]
        for k in range(1, 7):
            ssum = ssum + y[k]
        y[7] = cpv[7] * 2.0 - ssum
        bs.append([_cround(jnp.fmod(y[j], Q)) for j in range(8)])
        xl = [cpv[j] * 0.25 for j in range(8)]
    xh = [jnp.zeros_like(xs[0]) for _ in range(8)]
    for i, b in enumerate(bs):
        pt = [None] * 8
        pt[0] = 2.0 * b[0]
        for k in range(1, 7):
            pt[k] = b[k] - b[k - 1]
        sb = b[0]
        for k in range(1, 8):
            sb = sb + b[k]
        pt[7] = 0.5 * sb
        u = [pt[j] * 0.25 for j in range(8)]
        c = _cp8(u)
        scale = Q ** i
        xh = [xh[j] + scale * (pt[j] - Q * c[j]) for j in range(8)]
    return xh


def _body(xf_hbm, beta_hbm, eps_hbm, out_hbm, in_v, out_v, beta_v, eps_v):
    wid = lax.axis_index("s") * 2 + lax.axis_index("c")
    base = wid * ROWS_PER_W * 8
    pltpu.sync_copy(xf_hbm.at[pl.ds(base, ROWS_PER_W * 8)], in_v)
    pltpu.sync_copy(beta_hbm, beta_v)
    pltpu.sync_copy(eps_hbm, eps_v)
    bv = beta_v[...]
    e = [eps_v[j] for j in range(8)]
    lanes = lax.iota(jnp.int32, 16)
    cols = [lanes * 8 + j for j in range(8)]

    def group(g, carry):
        goff = g * 128
        idx = [cols[j] + goff for j in range(8)]
        xs = [plsc.load_gather(in_v, [idx[j]]) / bv for j in range(8)]
        xh = _quantize16(xs, e)
        for j in range(8):
            plsc.store_scatter(out_v, [idx[j]], xh[j] * bv)
        return carry

    lax.fori_loop(0, GROUPS, group, 0)
    pltpu.sync_copy(out_v, out_hbm.at[pl.ds(base, ROWS_PER_W * 8)])


def kernel(x, beta, G, eps):
    orig = x.shape
    xf = x.reshape(-1)
    beta_b = jnp.broadcast_to(jnp.asarray(beta, jnp.float32).reshape(1), (16,))
    eps_b = jnp.broadcast_to(jnp.asarray(eps, jnp.float32).reshape(8, 1),
                             (8, 16))
    mesh = plsc.VectorSubcoreMesh(core_axis_name="c", subcore_axis_name="s")
    run = functools.partial(
        pl.kernel,
        out_type=jax.ShapeDtypeStruct((N_ROWS * 8,), jnp.float32),
        mesh=mesh,
        compiler_params=pltpu.CompilerParams(needs_layout_passes=False),
        scratch_types=[
            pltpu.VMEM((ROWS_PER_W * 8,), jnp.float32),
            pltpu.VMEM((ROWS_PER_W * 8,), jnp.float32),
            pltpu.VMEM((16,), jnp.float32),
            pltpu.VMEM((8, 16), jnp.float32),
        ],
    )(_body)
    out = run(xf, beta_b, eps_b)
    return out.reshape(orig)
